# baseline (speedup 1.0000x reference)
import functools

import jax
import jax.numpy as jnp
from jax import lax
from jax.experimental import pallas as pl
from jax.experimental.pallas import tpu as pltpu
from jax.experimental.pallas import tpu_sc as plsc

B, T, V, H = 4, 2048, 1000000, 128
K1, K2 = 6, 2
NT = B * T

_NC, _NS = 2, 16
_NW = _NC * _NS
_RPW = NT // _NW
_CH = 128
_NCH = _RPW // _CH


def _sc_gather(embed, idx2d):
    mesh = plsc.VectorSubcoreMesh(core_axis_name="c", subcore_axis_name="s")

    @functools.partial(
        pl.kernel,
        mesh=mesh,
        out_type=jax.ShapeDtypeStruct((NT, H), jnp.float32),
        scratch_types=[
            pltpu.VMEM((_NCH, _CH), jnp.int32),
            pltpu.VMEM((_RPW, H), jnp.float32),
            pltpu.SemaphoreType.DMA,
        ],
    )
    def k(tab_hbm, idx_hbm, out_hbm, idx_v, rows_v, sem):
        wid = lax.axis_index("s") * _NC + lax.axis_index("c")
        pltpu.sync_copy(idx_hbm.at[pl.ds(wid * _NCH, _NCH)], idx_v)
        cps = [
            pltpu.async_copy(
                tab_hbm.at[idx_v.at[j]],
                rows_v.at[pl.ds(j * _CH, _CH)],
                sem,
            )
            for j in range(_NCH)
        ]
        for c in cps:
            c.wait()
        pltpu.sync_copy(rows_v, out_hbm.at[pl.ds(wid * _RPW, _RPW)])

    return k(embed, idx2d)


def _mid_body(h0_ref, W1_ref, b1_ref, W2_ref, b2_ref, g_ref, be_ref,
              wg1_ref, wg2_ref, Wq_ref, bq_ref, ctx_ref):
    x0 = h0_ref[...]
    a = jnp.dot(x0, W1_ref[...], preferred_element_type=jnp.float32)
    a = jnp.maximum(a + b1_ref[...][None, :], 0.0)
    f = jnp.dot(a, W2_ref[...], preferred_element_type=jnp.float32)
    y = x0 + f + b2_ref[...][None, :]
    mu = jnp.mean(y, axis=1, keepdims=True)
    d = y - mu
    var = jnp.mean(d * d, axis=1, keepdims=True)
    h = d * lax.rsqrt(var + 1e-5) * g_ref[...][None, :] + be_ref[...][None, :]

    g1 = jnp.dot(h, wg1_ref[...], preferred_element_type=jnp.float32)
    g2 = jnp.dot(h, wg2_ref[...], preferred_element_type=jnp.float32)
    iota = lax.broadcasted_iota(jnp.int32, (T, 1), 0)
    NEG = jnp.float32(-1e30)

    sel = jnp.zeros((T, 1), jnp.bool_)
    cur = g1
    for _ in range(K1):
        mx = jnp.max(cur, axis=0, keepdims=True)
        first = jnp.min(jnp.where(cur == mx, iota, T), axis=0, keepdims=True)
        new = iota == first
        sel = jnp.logical_or(sel, new)
        cur = jnp.where(new, NEG, cur)
    cur = jnp.where(sel, NEG, g2)
    for _ in range(K2):
        mx = jnp.max(cur, axis=0, keepdims=True)
        first = jnp.min(jnp.where(cur == mx, iota, T), axis=0, keepdims=True)
        new = iota == first
        sel = jnp.logical_or(sel, new)
        cur = jnp.where(new, NEG, cur)

    q = jnp.dot(h[T - 1:T, :], Wq_ref[...],
                preferred_element_type=jnp.float32) + bq_ref[...][None, :]
    s = jnp.sum(h * q, axis=1, keepdims=True)
    mx = jnp.max(jnp.where(sel, s, NEG), axis=0, keepdims=True)
    w = jnp.where(sel, jnp.exp(s - mx), 0.0)
    w = w / jnp.sum(w, axis=0, keepdims=True)
    ctx_ref[...] = jnp.sum(w * h, axis=0, keepdims=True)[None]


def _mid(h0, W1, b1, W2, b2, gamma, beta, Wg1, Wg2h, Wq, bq):
    full = lambda *shape: pl.BlockSpec(shape, lambda bb: (0,) * len(shape))
    return pl.pallas_call(
        _mid_body,
        grid=(B,),
        in_specs=[
            pl.BlockSpec((T, H), lambda bb: (bb, 0)),
            full(H, 2 * H), full(2 * H), full(2 * H, H), full(H),
            full(H), full(H), full(H, 1), full(H, 1), full(H, H), full(H),
        ],
        out_specs=pl.BlockSpec((1, 1, H), lambda bb: (bb, 0, 0)),
        out_shape=jax.ShapeDtypeStruct((B, 1, H), jnp.float32),
    )(h0, W1, b1, W2, b2, gamma, beta, Wg1, Wg2h, Wq, bq).reshape(B, H)


_VT = 2048


def _proj_body(ctx_ref, wo_ref, bo_ref, out_ref):
    out_ref[...] = jnp.dot(
        ctx_ref[...], wo_ref[...], preferred_element_type=jnp.float32
    ) + bo_ref[...][None, :]


def _proj(ctx, Wo, bo):
    return pl.pallas_call(
        _proj_body,
        grid=(pl.cdiv(V, _VT),),
        in_specs=[
            pl.BlockSpec((B, H), lambda j: (0, 0)),
            pl.BlockSpec((H, _VT), lambda j: (0, j)),
            pl.BlockSpec((_VT,), lambda j: (j,)),
        ],
        out_specs=pl.BlockSpec((B, _VT), lambda j: (0, j)),
        out_shape=jax.ShapeDtypeStruct((B, V), jnp.float32),
    )(ctx, Wo, bo)


def kernel(seq, embed, W1, b1, W2, b2, gamma, beta, Wg1, bg1, Wg2, bg2,
           Wq, bq, Wo, bo):
    idx2d = seq.astype(jnp.int32).reshape(NT // _CH, _CH)
    h0 = _sc_gather(embed, idx2d)
    ctx = _mid(h0, W1, b1, W2, b2, gamma, beta, Wg1, Wg2[:H], Wq, bq)
    return _proj(ctx, Wo, bo)


# speedup vs baseline: 1.3052x; 1.3052x over previous
import functools

import jax
import jax.numpy as jnp
from jax import lax
from jax.experimental import pallas as pl
from jax.experimental.pallas import tpu as pltpu
from jax.experimental.pallas import tpu_sc as plsc

B, T, V, H = 4, 2048, 1000000, 128
K1, K2 = 6, 2
NT = B * T

_NC, _NS = 2, 16
_NW = _NC * _NS
_RPW = NT // _NW
_CH = 128
_NCH = _RPW // _CH


def _sc_gather(embed, idx2d):
    mesh = plsc.VectorSubcoreMesh(core_axis_name="c", subcore_axis_name="s")

    @functools.partial(
        pl.kernel,
        mesh=mesh,
        out_type=jax.ShapeDtypeStruct((NT, H), jnp.float32),
        scratch_types=[
            pltpu.VMEM((_NCH, _CH), jnp.int32),
            pltpu.VMEM((_RPW, H), jnp.float32),
            pltpu.SemaphoreType.DMA,
        ],
    )
    def k(tab_hbm, idx_hbm, out_hbm, idx_v, rows_v, sem):
        wid = lax.axis_index("s") * _NC + lax.axis_index("c")
        pltpu.sync_copy(idx_hbm.at[pl.ds(wid * _NCH, _NCH)], idx_v)
        cps = [
            pltpu.async_copy(
                tab_hbm.at[idx_v.at[j]],
                rows_v.at[pl.ds(j * _CH, _CH)],
                sem,
            )
            for j in range(_NCH)
        ]
        for c in cps:
            c.wait()
        pltpu.sync_copy(rows_v, out_hbm.at[pl.ds(wid * _RPW, _RPW)])

    return k(embed, idx2d)


def _mid_body(h0_ref, W1_ref, b1_ref, W2_ref, b2_ref, g_ref, be_ref,
              wg1_ref, wg2_ref, Wq_ref, bq_ref, ctx_ref):
    x0 = h0_ref[...]
    a = jnp.dot(x0, W1_ref[...], preferred_element_type=jnp.float32)
    a = jnp.maximum(a + b1_ref[...][None, :], 0.0)
    f = jnp.dot(a, W2_ref[...], preferred_element_type=jnp.float32)
    y = x0 + f + b2_ref[...][None, :]
    mu = jnp.mean(y, axis=1, keepdims=True)
    d = y - mu
    var = jnp.mean(d * d, axis=1, keepdims=True)
    h = d * lax.rsqrt(var + 1e-5) * g_ref[...][None, :] + be_ref[...][None, :]

    g1 = jnp.dot(h, wg1_ref[...], preferred_element_type=jnp.float32)
    g2 = jnp.dot(h, wg2_ref[...], preferred_element_type=jnp.float32)
    iota = lax.broadcasted_iota(jnp.int32, (T, 1), 0)
    NEG = jnp.float32(-1e30)

    sel = jnp.zeros((T, 1), jnp.bool_)
    cur = g1
    for _ in range(K1):
        mx = jnp.max(cur, axis=0, keepdims=True)
        first = jnp.min(jnp.where(cur == mx, iota, T), axis=0, keepdims=True)
        new = iota == first
        sel = jnp.logical_or(sel, new)
        cur = jnp.where(new, NEG, cur)
    cur = jnp.where(sel, NEG, g2)
    for _ in range(K2):
        mx = jnp.max(cur, axis=0, keepdims=True)
        first = jnp.min(jnp.where(cur == mx, iota, T), axis=0, keepdims=True)
        new = iota == first
        sel = jnp.logical_or(sel, new)
        cur = jnp.where(new, NEG, cur)

    q = jnp.dot(h[T - 1:T, :], Wq_ref[...],
                preferred_element_type=jnp.float32) + bq_ref[...][None, :]
    s = jnp.sum(h * q, axis=1, keepdims=True)
    mx = jnp.max(jnp.where(sel, s, NEG), axis=0, keepdims=True)
    w = jnp.where(sel, jnp.exp(s - mx), 0.0)
    w = w / jnp.sum(w, axis=0, keepdims=True)
    ctx_ref[...] = jnp.sum(w * h, axis=0, keepdims=True)[None]


def _mid(h0, W1, b1, W2, b2, gamma, beta, Wg1, Wg2h, Wq, bq):
    full = lambda *shape: pl.BlockSpec(shape, lambda bb: (0,) * len(shape))
    return pl.pallas_call(
        _mid_body,
        grid=(B,),
        in_specs=[
            pl.BlockSpec((T, H), lambda bb: (bb, 0)),
            full(H, 2 * H), full(2 * H), full(2 * H, H), full(H),
            full(H), full(H), full(H, 1), full(H, 1), full(H, H), full(H),
        ],
        out_specs=pl.BlockSpec((1, 1, H), lambda bb: (bb, 0, 0)),
        out_shape=jax.ShapeDtypeStruct((B, 1, H), jnp.float32),
    )(h0, W1, b1, W2, b2, gamma, beta, Wg1, Wg2h, Wq, bq).reshape(B, H)


_VT = 16384


def _proj_body(ctx_ref, wo_ref, bo_ref, out_ref):
    out_ref[...] = jnp.dot(
        ctx_ref[...], wo_ref[...], preferred_element_type=jnp.float32
    ) + bo_ref[...][None, :]


def _proj(ctx, Wo, bo):
    return pl.pallas_call(
        _proj_body,
        grid=(pl.cdiv(V, _VT),),
        in_specs=[
            pl.BlockSpec((B, H), lambda j: (0, 0)),
            pl.BlockSpec((H, _VT), lambda j: (0, j)),
            pl.BlockSpec((_VT,), lambda j: (j,)),
        ],
        out_specs=pl.BlockSpec((B, _VT), lambda j: (0, j)),
        out_shape=jax.ShapeDtypeStruct((B, V), jnp.float32),
    )(ctx, Wo, bo)


def kernel(seq, embed, W1, b1, W2, b2, gamma, beta, Wg1, bg1, Wg2, bg2,
           Wq, bq, Wo, bo):
    idx2d = seq.astype(jnp.int32).reshape(NT // _CH, _CH)
    h0 = _sc_gather(embed, idx2d)
    ctx = _mid(h0, W1, b1, W2, b2, gamma, beta, Wg1, Wg2[:H], Wq, bq)
    return _proj(ctx, Wo, bo)
